# baseline (speedup 1.0000x reference)
import functools

import jax
import jax.numpy as jnp
from jax import lax
from jax.experimental import pallas as pl
from jax.experimental.pallas import tpu as pltpu
from jax.experimental.pallas import tpu_sc as plsc

_N = 10000
_E = 320000
_D = 128
_H = 128
_NC = 2
_NS = 16
_NW = _NC * _NS
_EPT = _E // _NW
_K = 80
_NCH = _EPT // _K
_NP = 10240
_RPT = _NP // _NS
_NPAD = 10112
_NRT = _NPAD // _NS
_BN = 1000

_mesh = plsc.VectorSubcoreMesh(
    core_axis_name="c", subcore_axis_name="s", num_cores=_NC, num_subcores=_NS
)


@functools.partial(
    pl.kernel,
    out_type=jax.ShapeDtypeStruct((_NC, _NP, _D), jnp.float32),
    mesh=_mesh,
    scratch_types=(
        [pltpu.VMEM((_K,), jnp.int32)] * 4
        + [
            pltpu.VMEM((_K, _D), jnp.float32),
            pltpu.VMEM((8, _D), jnp.float32),
            pltpu.VMEM_SHARED((_NP, _D), jnp.float32),
        ]
        + [pltpu.SemaphoreType.DMA] * 6
    ),
)
def _deg_kernel(
    dst_hbm, deg_out, didx0, didx1, didx2, didx3, onesb, zb, accum,
    isem0, isem1, isem2, isem3, ssem, zsem,
):
    c = lax.axis_index("c")
    s = lax.axis_index("s")
    w = c * _NS + s
    zero16 = jnp.zeros((16,), jnp.float32)
    one16 = jnp.ones((16,), jnp.float32)

    def fo(r, carry):
        def foc(q, inner):
            onesb[r, pl.ds(q * 16, 16)] = one16
            return inner

        lax.fori_loop(0, _D // 16, foc, 0)
        return carry

    lax.fori_loop(0, _K, fo, 0)

    def fz(r, carry):
        def fzc(q, inner):
            zb[r, pl.ds(q * 16, 16)] = zero16
            return inner

        lax.fori_loop(0, _D // 16, fzc, 0)
        return carry

    lax.fori_loop(0, 8, fz, 0)

    def zs(q, carry):
        pltpu.async_copy(zb, accum.at[pl.ds(s * _RPT + q * 8, 8)], zsem)
        return carry

    lax.fori_loop(0, _RPT // 8, zs, 0)

    def zw(q, carry):
        pltpu.make_async_copy(zb, accum.at[pl.ds(s * _RPT, 8)], zsem).wait()
        return carry

    lax.fori_loop(0, _RPT // 8, zw, 0)
    plsc.subcore_barrier()

    dbufs = ((didx0, isem0), (didx1, isem1), (didx2, isem2), (didx3, isem3))

    def load_d(j, b):
        pltpu.async_copy(
            dst_hbm.at[pl.ds(w * _EPT + j * _K, _K)], dbufs[b][0], dbufs[b][1]
        )

    def step(j, b, drain, prefetch):
        db, sem = dbufs[b]
        pltpu.make_async_copy(dst_hbm.at[pl.ds(0, _K)], db, sem).wait()
        pltpu.async_copy(onesb, accum.at[db], ssem, add=True)
        if drain:
            pltpu.make_async_copy(onesb, accum.at[db], ssem).wait()
        if prefetch:
            load_d(j + 2, (b + 2) % 4)

    load_d(0, 0)
    load_d(1, 1)
    step(0, 0, False, True)
    step(1, 1, False, True)

    def body(g, carry):
        for t in range(4):
            j = 4 * g + 2 + t
            step(j, (2 + t) % 4, True, True)
        return carry

    lax.fori_loop(0, (_NCH - 5) // 4, body, 0)
    step(_NCH - 3, 2, True, True)
    step(_NCH - 2, 3, True, False)
    step(_NCH - 1, 0, True, False)
    pltpu.make_async_copy(onesb, accum.at[didx0], ssem).wait()
    pltpu.make_async_copy(onesb, accum.at[didx0], ssem).wait()
    plsc.subcore_barrier()
    pltpu.sync_copy(
        accum.at[pl.ds(s * _RPT, _RPT)], deg_out.at[c, pl.ds(s * _RPT, _RPT)]
    )


@functools.partial(
    pl.kernel,
    out_type=jax.ShapeDtypeStruct((_NC, _NPAD, _D), jnp.float32),
    mesh=_mesh,
    scratch_types=(
        [pltpu.VMEM((_K,), jnp.int32)] * 8
        + [
            pltpu.VMEM((_K, _D), jnp.float32),
            pltpu.VMEM((_K, _D), jnp.float32),
            pltpu.VMEM((8, _D), jnp.float32),
            pltpu.VMEM_SHARED((_NPAD, _D), jnp.float32),
        ]
        + [pltpu.SemaphoreType.DMA] * 7
    ),
)
def _agg_kernel(
    hs_hbm, src_hbm, dst_hbm, out_hbm,
    sidx0, didx0, sidx1, didx1, sidx2, didx2, sidx3, didx3,
    rows0, rows1, zb, accum,
    isem0, isem1, isem2, isem3, gsem0, gsem1, zsem,
):
    c = lax.axis_index("c")
    s = lax.axis_index("s")
    w = c * _NS + s
    zero16 = jnp.zeros((16,), jnp.float32)

    def fz(r, carry):
        def fzc(q, inner):
            zb[r, pl.ds(q * 16, 16)] = zero16
            return inner

        lax.fori_loop(0, _D // 16, fzc, 0)
        return carry

    lax.fori_loop(0, 8, fz, 0)

    def zs(q, carry):
        pltpu.async_copy(zb, accum.at[pl.ds(s * _NRT + q * 8, 8)], zsem)
        return carry

    lax.fori_loop(0, _NRT // 8, zs, 0)

    def zw(q, carry):
        pltpu.make_async_copy(zb, accum.at[pl.ds(s * _NRT, 8)], zsem).wait()
        return carry

    lax.fori_loop(0, _NRT // 8, zw, 0)
    plsc.subcore_barrier()

    def load_idx(j, sb, db, sem):
        pltpu.async_copy(src_hbm.at[w, j], sb, sem)
        pltpu.async_copy(dst_hbm.at[w, j], db, sem)

    def wait_idx(sb, db, sem):
        pltpu.make_async_copy(src_hbm.at[w, 0], sb, sem).wait()
        pltpu.make_async_copy(dst_hbm.at[w, 0], db, sem).wait()

    ibufs = (
        (sidx0, didx0, isem0),
        (sidx1, didx1, isem1),
        (sidx2, didx2, isem2),
        (sidx3, didx3, isem3),
    )
    gbufs = ((rows0, gsem0), (rows1, gsem1))

    for b in range(4):
        load_idx(b, *ibufs[b])
    wait_idx(*ibufs[0])
    pltpu.async_copy(hs_hbm.at[sidx0], rows0, gsem0)

    def body(g, carry):
        for t in range(4):
            j = 4 * g + 1 + t
            sb, db, sem = ibufs[(1 + t) % 4]
            grow, gsem = gbufs[(1 + t) % 2]
            prow, psem = gbufs[t % 2]
            pdb = ibufs[t % 4][1]
            wait_idx(sb, db, sem)
            pltpu.async_copy(hs_hbm.at[sb], grow, gsem)
            pltpu.make_async_copy(hs_hbm.at[sb], prow, psem).wait()
            pltpu.sync_copy(prow, accum.at[pdb], add=True)

            @pl.when(j + 3 <= _NCH - 1)
            def _():
                load_idx(j + 3, *ibufs[t % 4])

        return carry

    lax.fori_loop(0, (_NCH - 1) // 4, body, 0)
    pltpu.make_async_copy(hs_hbm.at[sidx0], rows0, gsem0).wait()
    pltpu.sync_copy(rows0, accum.at[didx0], add=True)
    plsc.subcore_barrier()
    pltpu.sync_copy(
        accum.at[pl.ds(s * _NRT, _NRT)], out_hbm.at[c, pl.ds(s * _NRT, _NRT)]
    )


def _mmscale_body(x_ref, w_ref, d_ref, o_ref):
    norm = lax.rsqrt(jnp.maximum(d_ref[0] + d_ref[1], 1.0))
    o_ref[...] = (
        jnp.dot(x_ref[...], w_ref[...], preferred_element_type=jnp.float32) * norm
    )


def _tc_mm_scale(x, wT, degp):
    return pl.pallas_call(
        _mmscale_body,
        grid=(_N // _BN,),
        in_specs=[
            pl.BlockSpec((_BN, _D), lambda i: (i, 0)),
            pl.BlockSpec((_D, _H), lambda i: (0, 0)),
            pl.BlockSpec((2, _BN, 1), lambda i: (0, i, 0)),
        ],
        out_specs=pl.BlockSpec((_BN, _H), lambda i: (i, 0)),
        out_shape=jax.ShapeDtypeStruct((_N, _H), jnp.float32),
    )(x, wT, degp)


def _post1_body(a_ref, d_ref, b_ref, o_ref):
    a = a_ref[0] + a_ref[1]
    norm = lax.rsqrt(jnp.maximum(d_ref[0] + d_ref[1], 1.0))
    h = jnp.maximum(norm * a + b_ref[...], 0.0)
    o_ref[...] = norm * h


def _tc_post1(agg, degp, b1r):
    return pl.pallas_call(
        _post1_body,
        grid=(_N // _BN,),
        in_specs=[
            pl.BlockSpec((2, _BN, _D), lambda i: (0, i, 0)),
            pl.BlockSpec((2, _BN, 1), lambda i: (0, i, 0)),
            pl.BlockSpec((1, _H), lambda i: (0, 0)),
        ],
        out_specs=pl.BlockSpec((_BN, _H), lambda i: (i, 0)),
        out_shape=jax.ShapeDtypeStruct((_N, _H), jnp.float32),
    )(agg, degp, b1r)


def _post2_body(a_ref, d_ref, w2_ref, b2_ref, wc_ref, bc_ref, o_ref, acc_ref):
    i = pl.program_id(0)
    a = a_ref[0] + a_ref[1]
    norm = lax.rsqrt(jnp.maximum(d_ref[0] + d_ref[1], 1.0))
    z = jnp.dot(norm * a, w2_ref[...], preferred_element_type=jnp.float32)
    h = jnp.maximum(z + b2_ref[...], 0.0)
    ssum = jnp.sum(h, axis=0, keepdims=True)

    @pl.when(i == 0)
    def _():
        acc_ref[...] = ssum

    @pl.when(i > 0)
    def _():
        acc_ref[...] = acc_ref[...] + ssum

    @pl.when(i == _N // _BN - 1)
    def _():
        hg = acc_ref[...] * (1.0 / _N)
        o_ref[...] = (
            jnp.dot(hg, wc_ref[...], preferred_element_type=jnp.float32)
            + bc_ref[...]
        )


def _tc_post2(agg, degp, w2T, b2r, wcT, bcr):
    return pl.pallas_call(
        _post2_body,
        grid=(_N // _BN,),
        in_specs=[
            pl.BlockSpec((2, _BN, _D), lambda i: (0, i, 0)),
            pl.BlockSpec((2, _BN, 1), lambda i: (0, i, 0)),
            pl.BlockSpec((_H, 256), lambda i: (0, 0)),
            pl.BlockSpec((1, 256), lambda i: (0, 0)),
            pl.BlockSpec((256, 10), lambda i: (0, 0)),
            pl.BlockSpec((1, 10), lambda i: (0, 0)),
        ],
        out_specs=pl.BlockSpec((1, 10), lambda i: (0, 0)),
        out_shape=jax.ShapeDtypeStruct((1, 10), jnp.float32),
        scratch_shapes=[pltpu.VMEM((1, 256), jnp.float32)],
    )(agg, degp, w2T, b2r, wcT, bcr)


def kernel(x, edge_index, W1, b1, W2, b2, Wc, bc):
    src3 = edge_index[0].reshape(_NW, _NCH, _K)
    dst3 = edge_index[1].reshape(_NW, _NCH, _K)
    degw = _deg_kernel(edge_index[1])
    degp = degw[:, :_N, :1]
    hs = _tc_mm_scale(x, W1.T, degp)
    a1 = _agg_kernel(hs, src3, dst3)
    h1s = _tc_post1(a1, degp, b1.reshape(1, _H))
    a2 = _agg_kernel(h1s, src3, dst3)
    y = _tc_post2(
        a2, degp, W2.T, b2.reshape(1, 256), Wc.T, bc.reshape(1, 10)
    )
    return y


# speedup vs baseline: 1.0035x; 1.0035x over previous
import functools

import jax
import jax.numpy as jnp
from jax import lax
from jax.experimental import pallas as pl
from jax.experimental.pallas import tpu as pltpu
from jax.experimental.pallas import tpu_sc as plsc

_N = 10000
_E = 320000
_D = 128
_H = 128
_NC = 2
_NS = 16
_NW = _NC * _NS
_EPT = _E // _NW
_K = 80
_NCH = _EPT // _K
_NP = 10240
_RPT = _NP // _NS
_NPAD = 10112
_NRT = _NPAD // _NS
_BN = 1000

_mesh = plsc.VectorSubcoreMesh(
    core_axis_name="c", subcore_axis_name="s", num_cores=_NC, num_subcores=_NS
)


@functools.partial(
    pl.kernel,
    out_type=jax.ShapeDtypeStruct((_NC, _NP, _D), jnp.float32),
    mesh=_mesh,
    scratch_types=(
        [pltpu.VMEM((_K,), jnp.int32)] * 4
        + [
            pltpu.VMEM((_K, _D), jnp.float32),
            pltpu.VMEM((8, _D), jnp.float32),
            pltpu.VMEM_SHARED((_NP, _D), jnp.float32),
        ]
        + [pltpu.SemaphoreType.DMA] * 6
    ),
)
def _deg_kernel(
    dst_hbm, deg_out, didx0, didx1, didx2, didx3, onesb, zb, accum,
    isem0, isem1, isem2, isem3, ssem, zsem,
):
    c = lax.axis_index("c")
    s = lax.axis_index("s")
    w = c * _NS + s
    zero16 = jnp.zeros((16,), jnp.float32)
    one16 = jnp.ones((16,), jnp.float32)

    def fo(r, carry):
        def foc(q, inner):
            onesb[r, pl.ds(q * 16, 16)] = one16
            return inner

        lax.fori_loop(0, _D // 16, foc, 0)
        return carry

    lax.fori_loop(0, _K, fo, 0)

    def fz(r, carry):
        def fzc(q, inner):
            zb[r, pl.ds(q * 16, 16)] = zero16
            return inner

        lax.fori_loop(0, _D // 16, fzc, 0)
        return carry

    lax.fori_loop(0, 8, fz, 0)

    def zs(q, carry):
        pltpu.async_copy(zb, accum.at[pl.ds(s * _RPT + q * 8, 8)], zsem)
        return carry

    lax.fori_loop(0, _RPT // 8, zs, 0)

    def zw(q, carry):
        pltpu.make_async_copy(zb, accum.at[pl.ds(s * _RPT, 8)], zsem).wait()
        return carry

    lax.fori_loop(0, _RPT // 8, zw, 0)
    plsc.subcore_barrier()

    dbufs = ((didx0, isem0), (didx1, isem1), (didx2, isem2), (didx3, isem3))

    def load_d(j, b):
        pltpu.async_copy(
            dst_hbm.at[pl.ds(w * _EPT + j * _K, _K)], dbufs[b][0], dbufs[b][1]
        )

    def step(j, b, drain, prefetch):
        db, sem = dbufs[b]
        pltpu.make_async_copy(dst_hbm.at[pl.ds(0, _K)], db, sem).wait()
        pltpu.async_copy(onesb, accum.at[db], ssem, add=True)
        if drain:
            pltpu.make_async_copy(onesb, accum.at[db], ssem).wait()
        if prefetch:
            load_d(j + 2, (b + 2) % 4)

    load_d(0, 0)
    load_d(1, 1)
    step(0, 0, False, True)
    step(1, 1, False, True)

    def body(g, carry):
        for t in range(4):
            j = 4 * g + 2 + t
            step(j, (2 + t) % 4, True, True)
        return carry

    lax.fori_loop(0, (_NCH - 5) // 4, body, 0)
    step(_NCH - 3, 2, True, True)
    step(_NCH - 2, 3, True, False)
    step(_NCH - 1, 0, True, False)
    pltpu.make_async_copy(onesb, accum.at[didx0], ssem).wait()
    pltpu.make_async_copy(onesb, accum.at[didx0], ssem).wait()
    plsc.subcore_barrier()
    pltpu.sync_copy(
        accum.at[pl.ds(s * _RPT, _RPT)], deg_out.at[c, pl.ds(s * _RPT, _RPT)]
    )


@functools.partial(
    pl.kernel,
    out_type=jax.ShapeDtypeStruct((_NC, _NPAD, _D), jnp.float32),
    mesh=_mesh,
    scratch_types=(
        [pltpu.VMEM((_K,), jnp.int32)] * 8
        + [
            pltpu.VMEM((_K, _D), jnp.float32),
            pltpu.VMEM((_K, _D), jnp.float32),
            pltpu.VMEM((8, _D), jnp.float32),
            pltpu.VMEM_SHARED((_NPAD, _D), jnp.float32),
        ]
        + [pltpu.SemaphoreType.DMA] * 7
    ),
)
def _agg_kernel(
    hs_hbm, src_hbm, dst_hbm, out_hbm,
    sidx0, didx0, sidx1, didx1, sidx2, didx2, sidx3, didx3,
    rows0, rows1, zb, accum,
    isem0, isem1, isem2, isem3, gsem0, gsem1, zsem,
):
    c = lax.axis_index("c")
    s = lax.axis_index("s")
    w = c * _NS + s
    zero16 = jnp.zeros((16,), jnp.float32)

    def fz(r, carry):
        def fzc(q, inner):
            zb[r, pl.ds(q * 16, 16)] = zero16
            return inner

        lax.fori_loop(0, _D // 16, fzc, 0)
        return carry

    lax.fori_loop(0, 8, fz, 0)

    def zs(q, carry):
        pltpu.async_copy(zb, accum.at[pl.ds(s * _NRT + q * 8, 8)], zsem)
        return carry

    lax.fori_loop(0, _NRT // 8, zs, 0)

    def zw(q, carry):
        pltpu.make_async_copy(zb, accum.at[pl.ds(s * _NRT, 8)], zsem).wait()
        return carry

    lax.fori_loop(0, _NRT // 8, zw, 0)
    plsc.subcore_barrier()

    def load_idx(j, sb, db, sem):
        pltpu.async_copy(src_hbm.at[w, j], sb, sem)
        pltpu.async_copy(dst_hbm.at[w, j], db, sem)

    def wait_idx(sb, db, sem):
        pltpu.make_async_copy(src_hbm.at[w, 0], sb, sem).wait()
        pltpu.make_async_copy(dst_hbm.at[w, 0], db, sem).wait()

    ibufs = (
        (sidx0, didx0, isem0),
        (sidx1, didx1, isem1),
        (sidx2, didx2, isem2),
        (sidx3, didx3, isem3),
    )
    gbufs = ((rows0, gsem0), (rows1, gsem1))

    for b in range(4):
        load_idx(b, *ibufs[b])
    wait_idx(*ibufs[0])
    pltpu.async_copy(hs_hbm.at[sidx0], rows0, gsem0)

    def body(g, carry):
        for t in range(4):
            j = 4 * g + 1 + t
            sb, db, sem = ibufs[(1 + t) % 4]
            grow, gsem = gbufs[(1 + t) % 2]
            prow, psem = gbufs[t % 2]
            pdb = ibufs[t % 4][1]
            wait_idx(sb, db, sem)
            pltpu.async_copy(hs_hbm.at[sb], grow, gsem)
            pltpu.make_async_copy(hs_hbm.at[sb], prow, psem).wait()
            pltpu.sync_copy(prow, accum.at[pdb], add=True)

            @pl.when(j + 3 <= _NCH - 1)
            def _():
                load_idx(j + 3, *ibufs[t % 4])

        return carry

    lax.fori_loop(0, (_NCH - 1) // 4, body, 0)
    pltpu.make_async_copy(hs_hbm.at[sidx0], rows0, gsem0).wait()
    pltpu.sync_copy(rows0, accum.at[didx0], add=True)
    plsc.subcore_barrier()
    pltpu.sync_copy(
        accum.at[pl.ds(s * _NRT, _NRT)], out_hbm.at[c, pl.ds(s * _NRT, _NRT)]
    )


def _mm_body(x_ref, w_ref, o_ref):
    o_ref[...] = jnp.dot(x_ref[...], w_ref[...], preferred_element_type=jnp.float32)


def _tc_mm(x, wT):
    return pl.pallas_call(
        _mm_body,
        grid=(_N // _BN,),
        in_specs=[
            pl.BlockSpec((_BN, _D), lambda i: (i, 0)),
            pl.BlockSpec((_D, _H), lambda i: (0, 0)),
        ],
        out_specs=pl.BlockSpec((_BN, _H), lambda i: (i, 0)),
        out_shape=jax.ShapeDtypeStruct((_N, _H), jnp.float32),
    )(x, wT)


def _scale_body(t_ref, d_ref, o_ref):
    norm = lax.rsqrt(jnp.maximum(d_ref[0] + d_ref[1], 1.0))
    o_ref[...] = t_ref[...] * norm


def _tc_scale(t1, degp):
    return pl.pallas_call(
        _scale_body,
        grid=(_N // _BN,),
        in_specs=[
            pl.BlockSpec((_BN, _H), lambda i: (i, 0)),
            pl.BlockSpec((2, _BN, 1), lambda i: (0, i, 0)),
        ],
        out_specs=pl.BlockSpec((_BN, _H), lambda i: (i, 0)),
        out_shape=jax.ShapeDtypeStruct((_N, _H), jnp.float32),
    )(t1, degp)


def _post1_body(a_ref, d_ref, b_ref, o_ref):
    a = a_ref[0] + a_ref[1]
    norm = lax.rsqrt(jnp.maximum(d_ref[0] + d_ref[1], 1.0))
    h = jnp.maximum(norm * a + b_ref[...], 0.0)
    o_ref[...] = norm * h


def _tc_post1(agg, degp, b1r):
    return pl.pallas_call(
        _post1_body,
        grid=(_N // _BN,),
        in_specs=[
            pl.BlockSpec((2, _BN, _D), lambda i: (0, i, 0)),
            pl.BlockSpec((2, _BN, 1), lambda i: (0, i, 0)),
            pl.BlockSpec((1, _H), lambda i: (0, 0)),
        ],
        out_specs=pl.BlockSpec((_BN, _H), lambda i: (i, 0)),
        out_shape=jax.ShapeDtypeStruct((_N, _H), jnp.float32),
    )(agg, degp, b1r)


def _post2_body(a_ref, d_ref, w2_ref, b2_ref, wc_ref, bc_ref, o_ref, acc_ref):
    i = pl.program_id(0)
    a = a_ref[0] + a_ref[1]
    norm = lax.rsqrt(jnp.maximum(d_ref[0] + d_ref[1], 1.0))
    z = jnp.dot(norm * a, w2_ref[...], preferred_element_type=jnp.float32)
    h = jnp.maximum(z + b2_ref[...], 0.0)
    ssum = jnp.sum(h, axis=0, keepdims=True)

    @pl.when(i == 0)
    def _():
        acc_ref[...] = ssum

    @pl.when(i > 0)
    def _():
        acc_ref[...] = acc_ref[...] + ssum

    @pl.when(i == _N // _BN - 1)
    def _():
        hg = acc_ref[...] * (1.0 / _N)
        o_ref[...] = (
            jnp.dot(hg, wc_ref[...], preferred_element_type=jnp.float32)
            + bc_ref[...]
        )


def _tc_post2(agg, degp, w2T, b2r, wcT, bcr):
    return pl.pallas_call(
        _post2_body,
        grid=(_N // _BN,),
        in_specs=[
            pl.BlockSpec((2, _BN, _D), lambda i: (0, i, 0)),
            pl.BlockSpec((2, _BN, 1), lambda i: (0, i, 0)),
            pl.BlockSpec((_H, 256), lambda i: (0, 0)),
            pl.BlockSpec((1, 256), lambda i: (0, 0)),
            pl.BlockSpec((256, 10), lambda i: (0, 0)),
            pl.BlockSpec((1, 10), lambda i: (0, 0)),
        ],
        out_specs=pl.BlockSpec((1, 10), lambda i: (0, 0)),
        out_shape=jax.ShapeDtypeStruct((1, 10), jnp.float32),
        scratch_shapes=[pltpu.VMEM((1, 256), jnp.float32)],
    )(agg, degp, w2T, b2r, wcT, bcr)


def kernel(x, edge_index, W1, b1, W2, b2, Wc, bc):
    src3 = edge_index[0].reshape(_NW, _NCH, _K)
    dst3 = edge_index[1].reshape(_NW, _NCH, _K)
    degw = _deg_kernel(edge_index[1])
    degp = degw[:, :_N, :1]
    t1 = _tc_mm(x, W1.T)
    hs = _tc_scale(t1, degp)
    a1 = _agg_kernel(hs, src3, dst3)
    h1s = _tc_post1(a1, degp, b1.reshape(1, _H))
    a2 = _agg_kernel(h1s, src3, dst3)
    y = _tc_post2(
        a2, degp, W2.T, b2.reshape(1, 256), Wc.T, bc.reshape(1, 10)
    )
    return y
